# baseline (device time: 67372 ns/iter reference)
import jax
import jax.numpy as jnp
from jax import lax
from jax.experimental import pallas as pl
from jax.experimental.pallas import tpu as pltpu

N_DEV = 8
B, SQ, H, D = 2, 128, 4, 64
HD = H * D
ROWS = B * SQ
CH = 2 * ROWS
NEG = -1e9


def kernel(x, Wq, K_ext, V_ext, Wo):
    def body(x_ref, wq_ref, k_ref, v_ref, wo_ref, out_ref,
             gath_ref, comm_ref, send_sems, recv_sems):
        my = lax.axis_index("i")
        left = lax.rem(my - 1 + N_DEV, N_DEV)
        right = lax.rem(my + 1, N_DEV)

        barrier_sem = pltpu.get_barrier_semaphore()
        for nbr in (left, right):
            pl.semaphore_signal(
                barrier_sem, inc=1,
                device_id=(nbr,), device_id_type=pl.DeviceIdType.MESH,
            )
        pl.semaphore_wait(barrier_sem, 2)

        kv = jnp.concatenate(
            [k_ref[...].reshape(ROWS, HD), v_ref[...].reshape(ROWS, HD)],
            axis=0,
        )
        gath_ref[pl.ds(my * CH, CH), :] = kv
        comm_ref[0, :, :] = kv

        for h in range(N_DEV - 1):
            send_slot = h % 2
            recv_slot = (h + 1) % 2
            rdma = pltpu.make_async_remote_copy(
                src_ref=comm_ref.at[send_slot],
                dst_ref=comm_ref.at[recv_slot],
                send_sem=send_sems.at[send_slot],
                recv_sem=recv_sems.at[recv_slot],
                device_id=(right,),
                device_id_type=pl.DeviceIdType.MESH,
            )
            rdma.start()
            rdma.wait()
            origin = lax.rem(my - h - 1 + N_DEV, N_DEV)
            gath_ref[pl.ds(origin * CH, CH), :] = comm_ref[recv_slot, :, :]

        x2 = x_ref[...].reshape(ROWS, 512)
        q2 = jnp.dot(x2, wq_ref[...], preferred_element_type=jnp.float32)

        ri = lax.broadcasted_iota(jnp.int32, (SQ, SQ), 0) // 64
        ci = lax.broadcasted_iota(jnp.int32, (SQ, SQ), 1) // 64
        blockdiag = ri == ci

        ctx_rows = []
        for b in range(B):
            ctx_heads = []
            for hh in range(H):
                q = q2[b * SQ:(b + 1) * SQ, hh * D:(hh + 1) * D]
                m = jnp.full((SQ, 1), NEG, dtype=jnp.float32)
                l = jnp.zeros((SQ, 1), dtype=jnp.float32)
                o = jnp.zeros((SQ, D), dtype=jnp.float32)
                for c in (0, 2, 4, 6):
                    kc = gath_ref[c * CH + b * SQ: c * CH + (b + 1) * SQ,
                                  hh * D:(hh + 1) * D]
                    vc = gath_ref[c * CH + ROWS + b * SQ:
                                  c * CH + ROWS + (b + 1) * SQ,
                                  hh * D:(hh + 1) * D]
                    s = lax.dot_general(
                        q, kc, (((1,), (1,)), ((), ())),
                        preferred_element_type=jnp.float32,
                    ) * 0.125
                    s = jnp.where(blockdiag, s, NEG)
                    m_c = jnp.max(s, axis=1, keepdims=True)
                    p = jnp.exp(s - m_c)
                    l_c = jnp.sum(p, axis=1, keepdims=True)
                    o_c = jnp.dot(p, vc, preferred_element_type=jnp.float32)
                    m_n = jnp.maximum(m, m_c)
                    sa = jnp.exp(m - m_n)
                    sc = jnp.exp(m_c - m_n)
                    l = l * sa + l_c * sc
                    o = o * sa + o_c * sc
                    m = m_n
                ctx_heads.append(o / l)
            ctx_rows.append(jnp.concatenate(ctx_heads, axis=1))
        ctx2 = jnp.concatenate(ctx_rows, axis=0)

        out2 = jnp.dot(ctx2, wo_ref[...], preferred_element_type=jnp.float32)
        out_ref[...] = out2.reshape(B, SQ, 512)

    return pl.pallas_call(
        body,
        out_shape=jax.ShapeDtypeStruct((B, SQ, 512), jnp.float32),
        in_specs=[pl.BlockSpec(memory_space=pltpu.VMEM)] * 5,
        out_specs=pl.BlockSpec(memory_space=pltpu.VMEM),
        scratch_shapes=[
            pltpu.VMEM((N_DEV * CH, HD), jnp.float32),
            pltpu.VMEM((2, CH, HD), jnp.float32),
            pltpu.SemaphoreType.DMA((2,)),
            pltpu.SemaphoreType.DMA((2,)),
        ],
        compiler_params=pltpu.CompilerParams(collective_id=0),
    )(x, Wq, K_ext, V_ext, Wo)


# device time: 23160 ns/iter; 2.9090x vs baseline; 2.9090x over previous
import jax
import jax.numpy as jnp
from jax import lax
from jax.experimental import pallas as pl
from jax.experimental.pallas import tpu as pltpu

N_DEV = 8
B, SQ, H, D = 2, 128, 4, 64
HD = H * D
ROWS = B * SQ
PROWS = ROWS + 8
NEG = -1e9


def kernel(x, Wq, K_ext, V_ext, Wo):
    def body(x_ref, wq_ref, k_ref, v_ref, wo_ref, out_ref,
             acc_ref, recv_ref, send_sems, recv_sems):
        my = lax.axis_index("i")
        p_x = jnp.bitwise_xor(my, 1)
        loc = lax.rem(my, 4)
        p_y = my - loc + (3 - loc)
        p_z = jnp.bitwise_xor(my, 4)
        partners = [p_x, p_y, p_z]

        barrier_sem = pltpu.get_barrier_semaphore()
        for nbr in partners:
            pl.semaphore_signal(
                barrier_sem, inc=1,
                device_id=(nbr,), device_id_type=pl.DeviceIdType.MESH,
            )
        pl.semaphore_wait(barrier_sem, 3)

        x2 = x_ref[...].reshape(ROWS, 512)
        q2 = jnp.dot(x2, wq_ref[...], preferred_element_type=jnp.float32)
        k2 = k_ref[...].reshape(ROWS, HD)
        v2 = v_ref[...].reshape(ROWS, HD)

        ri = lax.broadcasted_iota(jnp.int32, (SQ, SQ), 0) // 64
        ci = lax.broadcasted_iota(jnp.int32, (SQ, SQ), 1) // 64
        blockdiag = ri == ci
        is_even = lax.rem(my, 2) == 0

        for b in range(B):
            for hh in range(H):
                q = q2[b * SQ:(b + 1) * SQ, hh * D:(hh + 1) * D]
                kc = k2[b * SQ:(b + 1) * SQ, hh * D:(hh + 1) * D]
                vc = v2[b * SQ:(b + 1) * SQ, hh * D:(hh + 1) * D]
                s_t = lax.dot_general(
                    kc, q, (((1,), (1,)), ((), ())),
                    preferred_element_type=jnp.float32,
                ) * 0.125
                p_t = jnp.exp(jnp.where(blockdiag, s_t, NEG))
                p_t = jnp.where(is_even, p_t, 0.0)
                l_row = jnp.sum(p_t, axis=0, keepdims=True)
                o_bh = lax.dot_general(
                    p_t, vc, (((0,), (0,)), ((), ())),
                    preferred_element_type=jnp.float32,
                )
                acc_ref[b * SQ:(b + 1) * SQ, hh * D:(hh + 1) * D] = o_bh
                j = ROWS + b * H + hh
                acc_ref[j:j + 1, 0:SQ] = l_row
                acc_ref[j:j + 1, SQ:HD] = jnp.zeros((1, SQ), jnp.float32)

        for s in range(3):
            rdma = pltpu.make_async_remote_copy(
                src_ref=acc_ref,
                dst_ref=recv_ref.at[s],
                send_sem=send_sems.at[s],
                recv_sem=recv_sems.at[s],
                device_id=(partners[s],),
                device_id_type=pl.DeviceIdType.MESH,
            )
            rdma.start()
            rdma.wait()
            acc_ref[...] = acc_ref[...] + recv_ref[s, :, :]

        o2 = acc_ref[0:ROWS, :]
        ctx_rows = []
        for b in range(B):
            ctx_heads = []
            for hh in range(H):
                j = ROWS + b * H + hh
                l_col = acc_ref[j:j + 1, 0:SQ].reshape(SQ, 1)
                ctx_heads.append(
                    o2[b * SQ:(b + 1) * SQ, hh * D:(hh + 1) * D] / l_col
                )
            ctx_rows.append(jnp.concatenate(ctx_heads, axis=1))
        ctx2 = jnp.concatenate(ctx_rows, axis=0)

        out2 = jnp.dot(ctx2, wo_ref[...], preferred_element_type=jnp.float32)
        out_ref[...] = out2.reshape(B, SQ, 512)

    return pl.pallas_call(
        body,
        out_shape=jax.ShapeDtypeStruct((B, SQ, 512), jnp.float32),
        in_specs=[pl.BlockSpec(memory_space=pltpu.VMEM)] * 5,
        out_specs=pl.BlockSpec(memory_space=pltpu.VMEM),
        scratch_shapes=[
            pltpu.VMEM((PROWS, HD), jnp.float32),
            pltpu.VMEM((3, PROWS, HD), jnp.float32),
            pltpu.SemaphoreType.DMA((3,)),
            pltpu.SemaphoreType.DMA((3,)),
        ],
        compiler_params=pltpu.CompilerParams(collective_id=0),
    )(x, Wq, K_ext, V_ext, Wo)


# device time: 18798 ns/iter; 3.5840x vs baseline; 1.2320x over previous
import jax
import jax.numpy as jnp
from jax import lax
from jax.experimental import pallas as pl
from jax.experimental.pallas import tpu as pltpu

N_DEV = 8
B, SQ, H, D = 2, 128, 4, 64
HD = H * D
ROWS = B * SQ
PROWS = ROWS + 8
NEG = -1e9


def kernel(x, Wq, K_ext, V_ext, Wo):
    def body(x_ref, wq_ref, k_ref, v_ref, wo_ref, out_ref,
             acc_ref, recv_ref, send_sems, recv_sems):
        my = lax.axis_index("i")
        p_x = jnp.bitwise_xor(my, 1)
        loc = lax.rem(my, 4)
        p_y = my - loc + (3 - loc)
        p_z = jnp.bitwise_xor(my, 4)
        partners = [p_x, p_y, p_z]

        barrier_sem = pltpu.get_barrier_semaphore()
        for nbr in partners:
            pl.semaphore_signal(
                barrier_sem, inc=1,
                device_id=(nbr,), device_id_type=pl.DeviceIdType.MESH,
            )
        pl.semaphore_wait(barrier_sem, 3)

        x2 = x_ref[...].reshape(ROWS, 512)
        q2 = jnp.dot(x2, wq_ref[...], preferred_element_type=jnp.float32)
        k2 = k_ref[...].reshape(ROWS, HD)
        v2 = v_ref[...].reshape(ROWS, HD)

        ri = lax.broadcasted_iota(jnp.int32, (SQ, SQ), 0) // 64
        ci = lax.broadcasted_iota(jnp.int32, (SQ, SQ), 1) // 64
        blockdiag = ri == ci
        is_even = lax.rem(my, 2) == 0

        for b in range(B):
            for hh in range(H):
                q = q2[b * SQ:(b + 1) * SQ, hh * D:(hh + 1) * D]
                kc = k2[b * SQ:(b + 1) * SQ, hh * D:(hh + 1) * D]
                vc = v2[b * SQ:(b + 1) * SQ, hh * D:(hh + 1) * D]
                s_t = lax.dot_general(
                    kc, q, (((1,), (1,)), ((), ())),
                    preferred_element_type=jnp.float32,
                ) * 0.125
                p_t = jnp.exp(jnp.where(blockdiag, s_t, NEG))
                p_t = jnp.where(is_even, p_t, 0.0)
                l_row = jnp.sum(p_t, axis=0, keepdims=True)
                o_bh = lax.dot_general(
                    p_t, vc, (((0,), (0,)), ((), ())),
                    preferred_element_type=jnp.float32,
                )
                acc_ref[b * SQ:(b + 1) * SQ, hh * D:(hh + 1) * D] = (
                    o_bh.astype(jnp.bfloat16)
                )
                j = ROWS + b * H + hh
                acc_ref[j:j + 1, 0:SQ] = l_row.astype(jnp.bfloat16)
                acc_ref[j:j + 1, SQ:HD] = jnp.zeros((1, SQ), jnp.bfloat16)

        for s in range(3):
            rdma = pltpu.make_async_remote_copy(
                src_ref=acc_ref,
                dst_ref=recv_ref.at[s],
                send_sem=send_sems.at[s],
                recv_sem=recv_sems.at[s],
                device_id=(partners[s],),
                device_id_type=pl.DeviceIdType.MESH,
            )
            rdma.start()
            rdma.wait()
            acc_ref[...] = acc_ref[...] + recv_ref[s, :, :]

        o2 = acc_ref[0:ROWS, :].astype(jnp.float32)
        ctx_rows = []
        for b in range(B):
            ctx_heads = []
            for hh in range(H):
                j = ROWS + b * H + hh
                l_col = acc_ref[j:j + 1, 0:SQ].astype(jnp.float32).reshape(SQ, 1)
                ctx_heads.append(
                    o2[b * SQ:(b + 1) * SQ, hh * D:(hh + 1) * D] / l_col
                )
            ctx_rows.append(jnp.concatenate(ctx_heads, axis=1))
        ctx2 = jnp.concatenate(ctx_rows, axis=0)

        out2 = jnp.dot(ctx2, wo_ref[...], preferred_element_type=jnp.float32)
        out_ref[...] = out2.reshape(B, SQ, 512)

    return pl.pallas_call(
        body,
        out_shape=jax.ShapeDtypeStruct((B, SQ, 512), jnp.float32),
        in_specs=[pl.BlockSpec(memory_space=pltpu.VMEM)] * 5,
        out_specs=pl.BlockSpec(memory_space=pltpu.VMEM),
        scratch_shapes=[
            pltpu.VMEM((PROWS, HD), jnp.bfloat16),
            pltpu.VMEM((3, PROWS, HD), jnp.bfloat16),
            pltpu.SemaphoreType.DMA((3,)),
            pltpu.SemaphoreType.DMA((3,)),
        ],
        compiler_params=pltpu.CompilerParams(collective_id=0),
    )(x, Wq, K_ext, V_ext, Wo)


# device time: 18719 ns/iter; 3.5991x vs baseline; 1.0042x over previous
import jax
import jax.numpy as jnp
from jax import lax
from jax.experimental import pallas as pl
from jax.experimental.pallas import tpu as pltpu

N_DEV = 8
B, SQ, H, D = 2, 128, 4, 64
HD = H * D
ROWS = B * SQ
PROWS = ROWS + 8
NEG = -1e9


def kernel(x, Wq, K_ext, V_ext, Wo):
    def body(x_ref, wq_ref, k_ref, v_ref, wo_ref, out_ref,
             acc_ref, recv_ref, send_sems, recv_sems):
        my = lax.axis_index("i")
        p_x = jnp.bitwise_xor(my, 1)
        loc = lax.rem(my, 4)
        p_y = my - loc + (3 - loc)
        p_z = jnp.bitwise_xor(my, 4)
        partners = [p_x, p_y, p_z]

        barrier_sem = pltpu.get_barrier_semaphore()
        for nbr in partners:
            pl.semaphore_signal(
                barrier_sem, inc=1,
                device_id=(nbr,), device_id_type=pl.DeviceIdType.MESH,
            )
        pl.semaphore_wait(barrier_sem, 3)

        x2 = x_ref[...].reshape(ROWS, 512).astype(jnp.bfloat16)
        q2 = jnp.dot(x2, wq_ref[...].astype(jnp.bfloat16),
                     preferred_element_type=jnp.float32)
        q2 = q2.astype(jnp.bfloat16)
        k2 = k_ref[...].reshape(ROWS, HD).astype(jnp.bfloat16)
        v2 = v_ref[...].reshape(ROWS, HD).astype(jnp.bfloat16)

        ri = lax.broadcasted_iota(jnp.int32, (SQ, SQ), 0) // 64
        ci = lax.broadcasted_iota(jnp.int32, (SQ, SQ), 1) // 64
        blockdiag = ri == ci
        is_even = lax.rem(my, 2) == 0

        for b in range(B):
            for hh in range(H):
                q = q2[b * SQ:(b + 1) * SQ, hh * D:(hh + 1) * D]
                kc = k2[b * SQ:(b + 1) * SQ, hh * D:(hh + 1) * D]
                vc = v2[b * SQ:(b + 1) * SQ, hh * D:(hh + 1) * D]
                s_t = lax.dot_general(
                    kc, q, (((1,), (1,)), ((), ())),
                    preferred_element_type=jnp.float32,
                ) * 0.125
                p_t = jnp.exp(jnp.where(blockdiag, s_t, NEG))
                p_t = jnp.where(is_even, p_t, 0.0)
                l_row = jnp.sum(p_t, axis=0, keepdims=True)
                o_bh = lax.dot_general(
                    p_t.astype(jnp.bfloat16), vc, (((0,), (0,)), ((), ())),
                    preferred_element_type=jnp.float32,
                )
                acc_ref[b * SQ:(b + 1) * SQ, hh * D:(hh + 1) * D] = (
                    o_bh.astype(jnp.bfloat16)
                )
                j = ROWS + b * H + hh
                acc_ref[j:j + 1, 0:SQ] = l_row.astype(jnp.bfloat16)
                acc_ref[j:j + 1, SQ:HD] = jnp.zeros((1, SQ), jnp.bfloat16)

        for s in range(3):
            rdma = pltpu.make_async_remote_copy(
                src_ref=acc_ref,
                dst_ref=recv_ref.at[s],
                send_sem=send_sems.at[s],
                recv_sem=recv_sems.at[s],
                device_id=(partners[s],),
                device_id_type=pl.DeviceIdType.MESH,
            )
            rdma.start()
            rdma.wait()
            acc_ref[...] = acc_ref[...] + recv_ref[s, :, :]

        o2 = acc_ref[0:ROWS, :].astype(jnp.float32)
        ctx_rows = []
        for b in range(B):
            ctx_heads = []
            for hh in range(H):
                j = ROWS + b * H + hh
                l_col = acc_ref[j:j + 1, 0:SQ].astype(jnp.float32).reshape(SQ, 1)
                ctx_heads.append(
                    o2[b * SQ:(b + 1) * SQ, hh * D:(hh + 1) * D] / l_col
                )
            ctx_rows.append(jnp.concatenate(ctx_heads, axis=1))
        ctx2 = jnp.concatenate(ctx_rows, axis=0)

        out2 = jnp.dot(ctx2.astype(jnp.bfloat16),
                       wo_ref[...].astype(jnp.bfloat16),
                       preferred_element_type=jnp.float32)
        out_ref[...] = out2.reshape(B, SQ, 512)

    return pl.pallas_call(
        body,
        out_shape=jax.ShapeDtypeStruct((B, SQ, 512), jnp.float32),
        in_specs=[pl.BlockSpec(memory_space=pltpu.VMEM)] * 5,
        out_specs=pl.BlockSpec(memory_space=pltpu.VMEM),
        scratch_shapes=[
            pltpu.VMEM((PROWS, HD), jnp.bfloat16),
            pltpu.VMEM((3, PROWS, HD), jnp.bfloat16),
            pltpu.SemaphoreType.DMA((3,)),
            pltpu.SemaphoreType.DMA((3,)),
        ],
        compiler_params=pltpu.CompilerParams(collective_id=0),
    )(x, Wq, K_ext, V_ext, Wo)


# device time: 16727 ns/iter; 4.0277x vs baseline; 1.1191x over previous
import jax
import jax.numpy as jnp
from jax import lax
from jax.experimental import pallas as pl
from jax.experimental.pallas import tpu as pltpu

N_DEV = 8
B, SQ, H, D = 2, 128, 4, 64
HD = H * D
ROWS = B * SQ
PR = 136
NEG = -1e9
BF = jnp.bfloat16


def kernel(x, Wq, K_ext, V_ext, Wo):
    def body(x_ref, wq_ref, k_ref, v_ref, wo_ref, out_ref,
             acc_a, acc_b, recv_a, recv_b,
             send_a, recv_sem_a, send_b, recv_sem_b):
        my = lax.axis_index("i")
        p_x = jnp.bitwise_xor(my, 1)
        loc = lax.rem(my, 4)
        p_y = my - loc + (3 - loc)
        p_z = jnp.bitwise_xor(my, 4)
        order_a = [p_x, p_y, p_z]
        order_b = [p_y, p_z, p_x]

        barrier_sem = pltpu.get_barrier_semaphore()
        for nbr in (p_x, p_y, p_z):
            pl.semaphore_signal(
                barrier_sem, inc=1,
                device_id=(nbr,), device_id_type=pl.DeviceIdType.MESH,
            )
        pl.semaphore_wait(barrier_sem, 3)

        x2 = x_ref[...].reshape(ROWS, 512).astype(BF)
        q2 = jnp.dot(x2, wq_ref[...].astype(BF),
                     preferred_element_type=jnp.float32).astype(BF)
        k2 = k_ref[...].reshape(ROWS, HD).astype(BF)
        v2 = v_ref[...].reshape(ROWS, HD).astype(BF)

        ri = lax.broadcasted_iota(jnp.int32, (SQ, SQ), 0) // 64
        ci = lax.broadcasted_iota(jnp.int32, (SQ, SQ), 1) // 64
        blockdiag = ri == ci
        is_even = lax.rem(my, 2) == 0

        def partial(b, acc):
            for hh in range(H):
                q = q2[b * SQ:(b + 1) * SQ, hh * D:(hh + 1) * D]
                kc = k2[b * SQ:(b + 1) * SQ, hh * D:(hh + 1) * D]
                vc = v2[b * SQ:(b + 1) * SQ, hh * D:(hh + 1) * D]
                s_t = lax.dot_general(
                    kc, q, (((1,), (1,)), ((), ())),
                    preferred_element_type=jnp.float32,
                ) * 0.125
                p_t = jnp.exp(jnp.where(blockdiag, s_t, NEG))
                p_t = jnp.where(is_even, p_t, 0.0)
                l_row = jnp.sum(p_t, axis=0, keepdims=True)
                o_bh = lax.dot_general(
                    p_t.astype(BF), vc, (((0,), (0,)), ((), ())),
                    preferred_element_type=jnp.float32,
                )
                acc[0:SQ, hh * D:(hh + 1) * D] = o_bh.astype(BF)
                acc[SQ + hh:SQ + hh + 1, 0:SQ] = l_row.astype(BF)
                acc[SQ + hh:SQ + hh + 1, SQ:HD] = jnp.zeros((1, SQ), BF)

        def stage(s, acc, recv, ssem, rsem, partner):
            return pltpu.make_async_remote_copy(
                src_ref=acc,
                dst_ref=recv.at[s],
                send_sem=ssem.at[s],
                recv_sem=rsem.at[s],
                device_id=(partner,),
                device_id_type=pl.DeviceIdType.MESH,
            )

        def finish(b, acc):
            o2 = acc[0:SQ, :].astype(jnp.float32)
            heads = []
            for hh in range(H):
                l_col = (acc[SQ + hh:SQ + hh + 1, 0:SQ]
                         .astype(jnp.float32).reshape(SQ, 1))
                heads.append(o2[:, hh * D:(hh + 1) * D] / l_col)
            ctx = jnp.concatenate(heads, axis=1)
            out_b = jnp.dot(ctx.astype(BF), wo_ref[...].astype(BF),
                            preferred_element_type=jnp.float32)
            out_ref[b, :, :] = out_b

        partial(0, acc_a)
        a0 = stage(0, acc_a, recv_a, send_a, recv_sem_a, order_a[0])
        a0.start()
        partial(1, acc_b)
        b0 = stage(0, acc_b, recv_b, send_b, recv_sem_b, order_b[0])
        b0.start()

        a0.wait()
        acc_a[...] = acc_a[...] + recv_a[0, :, :]
        a1 = stage(1, acc_a, recv_a, send_a, recv_sem_a, order_a[1])
        a1.start()
        b0.wait()
        acc_b[...] = acc_b[...] + recv_b[0, :, :]
        b1 = stage(1, acc_b, recv_b, send_b, recv_sem_b, order_b[1])
        b1.start()

        a1.wait()
        acc_a[...] = acc_a[...] + recv_a[1, :, :]
        a2 = stage(2, acc_a, recv_a, send_a, recv_sem_a, order_a[2])
        a2.start()
        b1.wait()
        acc_b[...] = acc_b[...] + recv_b[1, :, :]
        b2 = stage(2, acc_b, recv_b, send_b, recv_sem_b, order_b[2])
        b2.start()

        a2.wait()
        acc_a[...] = acc_a[...] + recv_a[2, :, :]
        finish(0, acc_a)
        b2.wait()
        acc_b[...] = acc_b[...] + recv_b[2, :, :]
        finish(1, acc_b)

    return pl.pallas_call(
        body,
        out_shape=jax.ShapeDtypeStruct((B, SQ, 512), jnp.float32),
        in_specs=[pl.BlockSpec(memory_space=pltpu.VMEM)] * 5,
        out_specs=pl.BlockSpec(memory_space=pltpu.VMEM),
        scratch_shapes=[
            pltpu.VMEM((PR, HD), BF),
            pltpu.VMEM((PR, HD), BF),
            pltpu.VMEM((3, PR, HD), BF),
            pltpu.VMEM((3, PR, HD), BF),
            pltpu.SemaphoreType.DMA((3,)),
            pltpu.SemaphoreType.DMA((3,)),
            pltpu.SemaphoreType.DMA((3,)),
            pltpu.SemaphoreType.DMA((3,)),
        ],
        compiler_params=pltpu.CompilerParams(collective_id=0),
    )(x, Wq, K_ext, V_ext, Wo)
